# baseline (speedup 1.0000x reference)
import functools

import numpy as np
import jax
import jax.numpy as jnp
from jax import lax
from jax.experimental import pallas as pl
from jax.experimental.pallas import tpu as pltpu
from jax.experimental.pallas import tpu_sc as plsc

_SCALE = 1.0
_L = 16
_F = 2
_LOG2_T = 19
_T = 2 ** _LOG2_T
_MASK = _T - 1
_N_MIN = 128
_MAX_RES = 512
_GROWTH = float(np.exp(np.log(_MAX_RES * _SCALE / _N_MIN) / (_L - 1)))
_N_PTS = 1048576
_P2 = 2654435761
_P3 = 805459861

_NC = 2
_NS = 16
_NW = _NC * _NS
_NP = _N_PTS // _NW
_C = 128
_CHUNKS = _NP // _C


def _pair_sum_body(t_ref, o_ref):
    s = t_ref[0, :, 0, :] + t_ref[0, :, 1, :]
    o_ref[...] = s.reshape(o_ref.shape)


_PS_BLK = 64


def _presum_table(tview):
    return pl.pallas_call(
        _pair_sum_body,
        grid=(_L, 4096 // _PS_BLK),
        in_specs=[pl.BlockSpec((1, _PS_BLK, 2, 128), lambda l, b: (l, b, 0, 0))],
        out_specs=pl.BlockSpec((_PS_BLK * 128,),
                               lambda l, b: (l * (4096 // _PS_BLK) + b,)),
        out_shape=jax.ShapeDtypeStruct((_L * _T,), jnp.float32),
    )(tview)


_CB = 8 * _C
_NSP = 3
_HALF3 = _NSP * _T // 2


def _sc_body(xs_hbm, ys_hbm, zs_hbm, tsum_hbm, sp3_hbm, s_hbm, out_hbm,
             xyzb, sall, idxb, shb, wb, gb, obuf, spt, sem):
    wid = lax.axis_index("s") * _NC + lax.axis_index("c")
    base = wid * _NP
    pltpu.sync_copy(s_hbm, sall)

    @pl.when(lax.axis_index("s") == 0)
    def _stage():
        pltpu.sync_copy(sp3_hbm, spt)

    plsc.subcore_barrier()

    p2 = jnp.uint32(_P2)
    p3 = jnp.uint32(_P3)
    mask = jnp.uint32(_MASK)
    zeros = jnp.zeros((16,), jnp.float32)
    himask = jnp.uint32(0xFFFF0000)

    def _pos_decode(pos):
        a = pos % 5
        m_not_sp = ((0 - a) >> 31) | ((10 - pos) >> 31)
        lvl_sp = pos // 5
        lvl_nsp = 3 + pos - (pos + 4) // 5
        lvl = lvl_sp ^ ((lvl_sp ^ lvl_nsp) & m_not_sp)
        return lvl, m_not_sp

    def issue(p, m_not_sp):
        @pl.when(m_not_sp == 0)
        def _sp():
            pltpu.async_copy(
                spt.at[idxb.at[pl.ds(p * _CB, _CB)]],
                gb.at[pl.ds(p * _CB, _CB)],
                sem.at[p])

        @pl.when(m_not_sp != 0)
        def _hbm():
            pltpu.async_copy(
                tsum_hbm.at[idxb.at[pl.ds(p * _CB, _CB)]],
                gb.at[pl.ds(p * _CB, _CB)],
                sem.at[p])

    def drain_accum(k_prev):
        pp = k_prev & 1
        _, pm_not_sp = _pos_decode(k_prev & 15)
        ocp = (k_prev >> 4) & 1
        pltpu.make_async_copy(
            tsum_hbm.at[pl.ds(0, _CB)],
            gb.at[pl.ds(pp * _CB, _CB)],
            sem.at[pp]).wait()

        @pl.when(pm_not_sp == 0)
        def _acc_sp():
            for v in range(_C // 16):
                a = obuf[pl.ds(ocp * _C + v * 16, 16)]
                for corner in range(8):
                    o = pp * _CB + corner * _C + v * 16
                    wu = lax.bitcast_convert_type(gb[pl.ds(o, 16)], jnp.uint32)
                    amt = shb[pl.ds(o, 16)].astype(jnp.uint32)
                    val = lax.bitcast_convert_type((wu << amt) & himask,
                                                   jnp.float32)
                    a = a + wb[pl.ds(o, 16)] * val
                obuf[pl.ds(ocp * _C + v * 16, 16)] = a

        @pl.when(pm_not_sp != 0)
        def _acc_hbm():
            for v in range(_C // 16):
                a = obuf[pl.ds(ocp * _C + v * 16, 16)]
                for corner in range(8):
                    o = pp * _CB + corner * _C + v * 16
                    a = a + wb[pl.ds(o, 16)] * gb[pl.ds(o, 16)]
                obuf[pl.ds(ocp * _C + v * 16, 16)] = a

    def step(k, carry):
        pos = k & 15
        lvl, m_not_sp = _pos_decode(pos)
        chunk = k >> 4
        p = k & 1
        cp = chunk & 1

        @pl.when(pos == 0)
        def _load_chunk():
            g0 = base + chunk * _C
            pltpu.sync_copy(xs_hbm.at[pl.ds(g0, _C)], xyzb.at[pl.ds(0, _C)])
            pltpu.sync_copy(ys_hbm.at[pl.ds(g0, _C)], xyzb.at[pl.ds(_C, _C)])
            pltpu.sync_copy(zs_hbm.at[pl.ds(g0, _C)], xyzb.at[pl.ds(2 * _C, _C)])
            for v in range(_C // 16):
                obuf[pl.ds(cp * _C + v * 16, 16)] = zeros

        sv = sall[pl.ds(lvl * 16, 16)]
        base_or = (jnp.full((16,), lvl, jnp.int32).astype(jnp.uint32)
                   << jnp.uint32(_LOG2_T))
        for v in range(_C // 16):
            xv = xyzb[pl.ds(v * 16, 16)]
            yv = xyzb[pl.ds(_C + v * 16, 16)]
            zv = xyzb[pl.ds(2 * _C + v * 16, 16)]
            px = ((xv + 1.0) * 0.5) * sv + 0.5
            py = ((yv + 1.0) * 0.5) * sv + 0.5
            pz = ((zv + 1.0) * 0.5) * sv + 0.5
            gx = px.astype(jnp.uint32)
            gy = py.astype(jnp.uint32)
            gz = pz.astype(jnp.uint32)
            wx1 = px - gx.astype(jnp.float32)
            wy1 = py - gy.astype(jnp.float32)
            wz1 = pz - gz.astype(jnp.float32)
            wx0 = 1.0 - wx1
            wy0 = 1.0 - wy1
            wz0 = 1.0 - wz1
            hy0 = gy * p2
            hy1 = hy0 + p2
            hz0 = gz * p3
            hz1 = hz0 + p3
            hxy = (gx ^ hy0, (gx + jnp.uint32(1)) ^ hy0,
                   gx ^ hy1, (gx + jnp.uint32(1)) ^ hy1)
            wxy = (wx0 * wy0, wx1 * wy0, wx0 * wy1, wx1 * wy1)
            mhbm = jnp.full((16,), m_not_sp, jnp.int32)
            for corner in range(8):
                hz = hz1 if (corner >> 2) & 1 else hz0
                wz = wz1 if (corner >> 2) & 1 else wz0
                eidx = (((hxy[corner & 3] ^ hz) & mask) | base_or)
                ei = eidx.astype(jnp.int32)
                mge = (jnp.int32(_HALF3 - 1) - ei) >> 31
                word = ei - (mge & jnp.int32(_HALF3))
                amt = jnp.int32(16) & ~mge
                idx = word ^ ((word ^ ei) & mhbm)
                o = p * _CB + corner * _C + v * 16
                idxb[pl.ds(o, 16)] = idx
                shb[pl.ds(o, 16)] = amt
                wb[pl.ds(o, 16)] = wxy[corner & 3] * wz
        issue(p, m_not_sp)

        @pl.when(k > 0)
        def _drain_prev():
            drain_accum(k - 1)

            @pl.when(pos == 0)
            def _flush_prev_chunk():
                pcp = (chunk - 1) & 1
                pltpu.sync_copy(
                    obuf.at[pl.ds(pcp * _C, _C)],
                    out_hbm.at[pl.ds(base + (chunk - 1) * _C, _C)])

        return carry

    total = _L * _CHUNKS
    lax.fori_loop(0, total, step, 0)
    drain_accum(total - 1)
    pltpu.sync_copy(
        obuf.at[pl.ds(((_CHUNKS - 1) & 1) * _C, _C)],
        out_hbm.at[pl.ds(base + (_CHUNKS - 1) * _C, _C)])


_sc_kernel = functools.partial(
    pl.kernel,
    out_type=jax.ShapeDtypeStruct((_N_PTS,), jnp.float32),
    mesh=plsc.VectorSubcoreMesh(core_axis_name="c", subcore_axis_name="s"),
    scratch_types=[
        pltpu.VMEM((_C * 3,), jnp.float32),
        pltpu.VMEM((_L * 16,), jnp.float32),
        pltpu.VMEM((2 * _CB,), jnp.int32),
        pltpu.VMEM((2 * _CB,), jnp.int32),
        pltpu.VMEM((2 * _CB,), jnp.float32),
        pltpu.VMEM((2 * _CB,), jnp.float32),
        pltpu.VMEM((2 * _C,), jnp.float32),
        pltpu.VMEM_SHARED((_HALF3,), jnp.float32),
        pltpu.SemaphoreType.DMA((2,)),
    ],
)(_sc_body)


def _pack3_body(a_ref, b_ref, o_ref):
    lo = lax.bitcast_convert_type(
        a_ref[...].astype(jnp.bfloat16), jnp.uint16).astype(jnp.uint32)
    hi = lax.bitcast_convert_type(
        b_ref[...].astype(jnp.bfloat16), jnp.uint16).astype(jnp.uint32)
    o_ref[...] = lax.bitcast_convert_type(lo | (hi << 16), jnp.float32)


def _pack3(tsum):
    blk = 32768
    nb = _HALF3 // blk
    return pl.pallas_call(
        _pack3_body,
        grid=(nb,),
        in_specs=[pl.BlockSpec((blk,), lambda i: (i,)),
                  pl.BlockSpec((blk,), lambda i: (i + nb,))],
        out_specs=pl.BlockSpec((blk,), lambda i: (i,)),
        out_shape=jax.ShapeDtypeStruct((_HALF3,), jnp.float32),
    )(tsum, tsum)


def _level_scales():
    s = [_N_MIN * (_GROWTH ** lvl) - 1.0 for lvl in range(_L)]
    return np.repeat(np.asarray(s, np.float32)[:, None], 16, axis=1).reshape(-1)


def kernel(x, table):
    tview = table.reshape(_L, _T // 128, 128, _F).transpose(0, 1, 3, 2)
    tsum = _presum_table(tview)
    s_splat = jnp.asarray(_level_scales())
    xt = x.T
    sp3 = _pack3(tsum)
    return _sc_kernel(xt[0], xt[1], xt[2], tsum, sp3, s_splat)


# speedup vs baseline: 1.1710x; 1.1710x over previous
import functools

import numpy as np
import jax
import jax.numpy as jnp
from jax import lax
from jax.experimental import pallas as pl
from jax.experimental.pallas import tpu as pltpu
from jax.experimental.pallas import tpu_sc as plsc

_SCALE = 1.0
_L = 16
_F = 2
_LOG2_T = 19
_T = 2 ** _LOG2_T
_MASK = _T - 1
_N_MIN = 128
_MAX_RES = 512
_GROWTH = float(np.exp(np.log(_MAX_RES * _SCALE / _N_MIN) / (_L - 1)))
_N_PTS = 1048576
_P2 = 2654435761
_P3 = 805459861

_NC = 2
_NS = 16
_NW = _NC * _NS
_NP = _N_PTS // _NW
_C = 128
_CHUNKS = _NP // _C


def _pair_sum_body(t_ref, o_ref):
    s = t_ref[0, :, 0, :] + t_ref[0, :, 1, :]
    o_ref[...] = s.reshape(o_ref.shape)


_PS_BLK = 64


def _presum_table(tview):
    return pl.pallas_call(
        _pair_sum_body,
        grid=(_L, 4096 // _PS_BLK),
        in_specs=[pl.BlockSpec((1, _PS_BLK, 2, 128), lambda l, b: (l, b, 0, 0))],
        out_specs=pl.BlockSpec((_PS_BLK * 128,),
                               lambda l, b: (l * (4096 // _PS_BLK) + b,)),
        out_shape=jax.ShapeDtypeStruct((_L * _T,), jnp.float32),
    )(tview)


_CB = 8 * _C
_NSP = 3
_HALF3 = _NSP * _T // 2


def _sc_body(xs_hbm, ys_hbm, zs_hbm, tsum_hbm, sp3_hbm, s_hbm, out_hbm,
             xyzb, sall, idxb, shb, wb, gb, obuf, spt, sem):
    wid = lax.axis_index("s") * _NC + lax.axis_index("c")
    base = wid * _NP
    pltpu.sync_copy(s_hbm, sall)

    @pl.when(lax.axis_index("s") == 0)
    def _stage():
        pltpu.sync_copy(sp3_hbm, spt)

    plsc.subcore_barrier()

    p2 = jnp.uint32(_P2)
    p3 = jnp.uint32(_P3)
    mask = jnp.uint32(_MASK)
    zeros = jnp.zeros((16,), jnp.float32)
    himask = jnp.uint32(0xFFFF0000)

    def _pos_decode(pos):
        a = pos % 5
        m_not_sp = ((0 - a) >> 31) | ((10 - pos) >> 31)
        lvl_sp = pos // 5
        lvl_nsp = 3 + pos - (pos + 4) // 5
        lvl = lvl_sp ^ ((lvl_sp ^ lvl_nsp) & m_not_sp)
        return lvl, m_not_sp

    def issue(p, m_not_sp):
        @pl.when(m_not_sp == 0)
        def _sp():
            pltpu.async_copy(
                spt.at[idxb.at[pl.ds(p * _CB, _CB)]],
                gb.at[pl.ds(p * _CB, _CB)],
                sem.at[p])

        @pl.when(m_not_sp != 0)
        def _hbm():
            pltpu.async_copy(
                tsum_hbm.at[idxb.at[pl.ds(p * _CB, _CB)]],
                gb.at[pl.ds(p * _CB, _CB)],
                sem.at[p])

    def drain_accum(k_prev):
        pp = k_prev & 3
        _, pm_not_sp = _pos_decode(k_prev & 15)
        ocp = (k_prev >> 4) & 1
        pltpu.make_async_copy(
            tsum_hbm.at[pl.ds(0, _CB)],
            gb.at[pl.ds(pp * _CB, _CB)],
            sem.at[pp]).wait()

        @pl.when(pm_not_sp == 0)
        def _acc_sp():
            for v in range(_C // 16):
                a = obuf[pl.ds(ocp * _C + v * 16, 16)]
                for corner in range(8):
                    o = pp * _CB + corner * _C + v * 16
                    wu = lax.bitcast_convert_type(gb[pl.ds(o, 16)], jnp.uint32)
                    amt = shb[pl.ds(o, 16)].astype(jnp.uint32)
                    val = lax.bitcast_convert_type((wu << amt) & himask,
                                                   jnp.float32)
                    a = a + wb[pl.ds(o, 16)] * val
                obuf[pl.ds(ocp * _C + v * 16, 16)] = a

        @pl.when(pm_not_sp != 0)
        def _acc_hbm():
            for v in range(_C // 16):
                a = obuf[pl.ds(ocp * _C + v * 16, 16)]
                for corner in range(8):
                    o = pp * _CB + corner * _C + v * 16
                    a = a + wb[pl.ds(o, 16)] * gb[pl.ds(o, 16)]
                obuf[pl.ds(ocp * _C + v * 16, 16)] = a

    def step(k, carry):
        pos = k & 15
        lvl, m_not_sp = _pos_decode(pos)
        chunk = k >> 4
        p = k & 3
        cp = chunk & 1

        @pl.when(pos == 0)
        def _load_chunk():
            g0 = base + chunk * _C
            pltpu.sync_copy(xs_hbm.at[pl.ds(g0, _C)], xyzb.at[pl.ds(0, _C)])
            pltpu.sync_copy(ys_hbm.at[pl.ds(g0, _C)], xyzb.at[pl.ds(_C, _C)])
            pltpu.sync_copy(zs_hbm.at[pl.ds(g0, _C)], xyzb.at[pl.ds(2 * _C, _C)])
            for v in range(_C // 16):
                obuf[pl.ds(cp * _C + v * 16, 16)] = zeros

        sv = sall[pl.ds(lvl * 16, 16)]
        base_or = (jnp.full((16,), lvl, jnp.int32).astype(jnp.uint32)
                   << jnp.uint32(_LOG2_T))
        for v in range(_C // 16):
            xv = xyzb[pl.ds(v * 16, 16)]
            yv = xyzb[pl.ds(_C + v * 16, 16)]
            zv = xyzb[pl.ds(2 * _C + v * 16, 16)]
            px = ((xv + 1.0) * 0.5) * sv + 0.5
            py = ((yv + 1.0) * 0.5) * sv + 0.5
            pz = ((zv + 1.0) * 0.5) * sv + 0.5
            gx = px.astype(jnp.uint32)
            gy = py.astype(jnp.uint32)
            gz = pz.astype(jnp.uint32)
            wx1 = px - gx.astype(jnp.float32)
            wy1 = py - gy.astype(jnp.float32)
            wz1 = pz - gz.astype(jnp.float32)
            wx0 = 1.0 - wx1
            wy0 = 1.0 - wy1
            wz0 = 1.0 - wz1
            hy0 = gy * p2
            hy1 = hy0 + p2
            hz0 = gz * p3
            hz1 = hz0 + p3
            hxy = (gx ^ hy0, (gx + jnp.uint32(1)) ^ hy0,
                   gx ^ hy1, (gx + jnp.uint32(1)) ^ hy1)
            wxy = (wx0 * wy0, wx1 * wy0, wx0 * wy1, wx1 * wy1)
            mhbm = jnp.full((16,), m_not_sp, jnp.int32)
            for corner in range(8):
                hz = hz1 if (corner >> 2) & 1 else hz0
                wz = wz1 if (corner >> 2) & 1 else wz0
                eidx = (((hxy[corner & 3] ^ hz) & mask) | base_or)
                ei = eidx.astype(jnp.int32)
                mge = (jnp.int32(_HALF3 - 1) - ei) >> 31
                word = ei - (mge & jnp.int32(_HALF3))
                amt = jnp.int32(16) & ~mge
                idx = word ^ ((word ^ ei) & mhbm)
                o = p * _CB + corner * _C + v * 16
                idxb[pl.ds(o, 16)] = idx
                shb[pl.ds(o, 16)] = amt
                wb[pl.ds(o, 16)] = wxy[corner & 3] * wz
        issue(p, m_not_sp)

        @pl.when(k > 2)
        def _drain_prev():
            drain_accum(k - 3)

            @pl.when(jnp.logical_and(pos == 3, chunk > 0))
            def _flush_prev_chunk():
                pcp = (chunk - 1) & 1
                pltpu.sync_copy(
                    obuf.at[pl.ds(pcp * _C, _C)],
                    out_hbm.at[pl.ds(base + (chunk - 1) * _C, _C)])

        return carry

    total = _L * _CHUNKS
    lax.fori_loop(0, total, step, 0)
    drain_accum(total - 3)
    drain_accum(total - 2)
    drain_accum(total - 1)
    pltpu.sync_copy(
        obuf.at[pl.ds(((_CHUNKS - 1) & 1) * _C, _C)],
        out_hbm.at[pl.ds(base + (_CHUNKS - 1) * _C, _C)])


_sc_kernel = functools.partial(
    pl.kernel,
    out_type=jax.ShapeDtypeStruct((_N_PTS,), jnp.float32),
    mesh=plsc.VectorSubcoreMesh(core_axis_name="c", subcore_axis_name="s"),
    scratch_types=[
        pltpu.VMEM((_C * 3,), jnp.float32),
        pltpu.VMEM((_L * 16,), jnp.float32),
        pltpu.VMEM((4 * _CB,), jnp.int32),
        pltpu.VMEM((4 * _CB,), jnp.int32),
        pltpu.VMEM((4 * _CB,), jnp.float32),
        pltpu.VMEM((4 * _CB,), jnp.float32),
        pltpu.VMEM((2 * _C,), jnp.float32),
        pltpu.VMEM_SHARED((_HALF3,), jnp.float32),
        pltpu.SemaphoreType.DMA((4,)),
    ],
)(_sc_body)


def _pack3_body(a_ref, b_ref, o_ref):
    lo = lax.bitcast_convert_type(
        a_ref[...].astype(jnp.bfloat16), jnp.uint16).astype(jnp.uint32)
    hi = lax.bitcast_convert_type(
        b_ref[...].astype(jnp.bfloat16), jnp.uint16).astype(jnp.uint32)
    o_ref[...] = lax.bitcast_convert_type(lo | (hi << 16), jnp.float32)


def _pack3(tsum):
    blk = 32768
    nb = _HALF3 // blk
    return pl.pallas_call(
        _pack3_body,
        grid=(nb,),
        in_specs=[pl.BlockSpec((blk,), lambda i: (i,)),
                  pl.BlockSpec((blk,), lambda i: (i + nb,))],
        out_specs=pl.BlockSpec((blk,), lambda i: (i,)),
        out_shape=jax.ShapeDtypeStruct((_HALF3,), jnp.float32),
    )(tsum, tsum)


def _level_scales():
    s = [_N_MIN * (_GROWTH ** lvl) - 1.0 for lvl in range(_L)]
    return np.repeat(np.asarray(s, np.float32)[:, None], 16, axis=1).reshape(-1)


def kernel(x, table):
    tview = table.reshape(_L, _T // 128, 128, _F).transpose(0, 1, 3, 2)
    tsum = _presum_table(tview)
    s_splat = jnp.asarray(_level_scales())
    xt = x.T
    sp3 = _pack3(tsum)
    return _sc_kernel(xt[0], xt[1], xt[2], tsum, sp3, s_splat)


# speedup vs baseline: 1.1766x; 1.0048x over previous
import functools

import numpy as np
import jax
import jax.numpy as jnp
from jax import lax
from jax.experimental import pallas as pl
from jax.experimental.pallas import tpu as pltpu
from jax.experimental.pallas import tpu_sc as plsc

_SCALE = 1.0
_L = 16
_F = 2
_LOG2_T = 19
_T = 2 ** _LOG2_T
_MASK = _T - 1
_N_MIN = 128
_MAX_RES = 512
_GROWTH = float(np.exp(np.log(_MAX_RES * _SCALE / _N_MIN) / (_L - 1)))
_N_PTS = 1048576
_P2 = 2654435761
_P3 = 805459861

_NC = 2
_NS = 16
_NW = _NC * _NS
_NP = _N_PTS // _NW
_C = 128
_CHUNKS = _NP // _C


def _pair_sum_body(t_ref, o_ref):
    s = t_ref[0, :, 0, :] + t_ref[0, :, 1, :]
    o_ref[...] = s.reshape(o_ref.shape)


_PS_BLK = 64


def _presum_table(tview):
    return pl.pallas_call(
        _pair_sum_body,
        grid=(_L, 4096 // _PS_BLK),
        in_specs=[pl.BlockSpec((1, _PS_BLK, 2, 128), lambda l, b: (l, b, 0, 0))],
        out_specs=pl.BlockSpec((_PS_BLK * 128,),
                               lambda l, b: (l * (4096 // _PS_BLK) + b,)),
        out_shape=jax.ShapeDtypeStruct((_L * _T,), jnp.float32),
    )(tview)


_CB = 8 * _C
_NSP = 3
_HALF3 = _NSP * _T // 2


def _sc_body(xs_hbm, ys_hbm, zs_hbm, tsum_hbm, sp3_hbm, s_hbm, out_hbm,
             xyzb, sall, idxb, shb, wb, gb, obuf, spt, sem):
    wid = lax.axis_index("s") * _NC + lax.axis_index("c")
    base = wid * _NP
    pltpu.sync_copy(s_hbm, sall)

    @pl.when(lax.axis_index("s") == 0)
    def _stage():
        pltpu.sync_copy(sp3_hbm, spt)

    plsc.subcore_barrier()

    p2 = jnp.uint32(_P2)
    p3 = jnp.uint32(_P3)
    mask = jnp.uint32(_MASK)
    zeros = jnp.zeros((16,), jnp.float32)
    himask = jnp.uint32(0xFFFF0000)

    def _pos_decode(pos):
        a = pos % 5
        m_not_sp = ((0 - a) >> 31) | ((10 - pos) >> 31)
        lvl_sp = pos // 5
        lvl_nsp = 3 + pos - (pos + 4) // 5
        lvl = lvl_sp ^ ((lvl_sp ^ lvl_nsp) & m_not_sp)
        return lvl, m_not_sp

    def issue(p, m_not_sp):
        @pl.when(m_not_sp == 0)
        def _sp():
            pltpu.async_copy(
                spt.at[idxb.at[pl.ds(p * _CB, _CB)]],
                gb.at[pl.ds(p * _CB, _CB)],
                sem.at[p])

        @pl.when(m_not_sp != 0)
        def _hbm():
            pltpu.async_copy(
                tsum_hbm.at[idxb.at[pl.ds(p * _CB, _CB)]],
                gb.at[pl.ds(p * _CB, _CB)],
                sem.at[p])

    def drain_accum(k_prev):
        pp = k_prev & 7
        _, pm_not_sp = _pos_decode(k_prev & 15)
        ocp = (k_prev >> 4) & 1
        pltpu.make_async_copy(
            tsum_hbm.at[pl.ds(0, _CB)],
            gb.at[pl.ds(pp * _CB, _CB)],
            sem.at[pp]).wait()

        @pl.when(pm_not_sp == 0)
        def _acc_sp():
            for v in range(_C // 16):
                a = obuf[pl.ds(ocp * _C + v * 16, 16)]
                for corner in range(8):
                    o = pp * _CB + corner * _C + v * 16
                    wu = lax.bitcast_convert_type(gb[pl.ds(o, 16)], jnp.uint32)
                    amt = shb[pl.ds(o, 16)].astype(jnp.uint32)
                    val = lax.bitcast_convert_type((wu << amt) & himask,
                                                   jnp.float32)
                    a = a + wb[pl.ds(o, 16)] * val
                obuf[pl.ds(ocp * _C + v * 16, 16)] = a

        @pl.when(pm_not_sp != 0)
        def _acc_hbm():
            for v in range(_C // 16):
                a = obuf[pl.ds(ocp * _C + v * 16, 16)]
                for corner in range(8):
                    o = pp * _CB + corner * _C + v * 16
                    a = a + wb[pl.ds(o, 16)] * gb[pl.ds(o, 16)]
                obuf[pl.ds(ocp * _C + v * 16, 16)] = a

    def step(k, carry):
        pos = k & 15
        lvl, m_not_sp = _pos_decode(pos)
        chunk = k >> 4
        p = k & 7
        cp = chunk & 1

        @pl.when(pos == 0)
        def _load_chunk():
            g0 = base + chunk * _C
            pltpu.sync_copy(xs_hbm.at[pl.ds(g0, _C)], xyzb.at[pl.ds(0, _C)])
            pltpu.sync_copy(ys_hbm.at[pl.ds(g0, _C)], xyzb.at[pl.ds(_C, _C)])
            pltpu.sync_copy(zs_hbm.at[pl.ds(g0, _C)], xyzb.at[pl.ds(2 * _C, _C)])
            for v in range(_C // 16):
                obuf[pl.ds(cp * _C + v * 16, 16)] = zeros

        sv = sall[pl.ds(lvl * 16, 16)]
        base_or = (jnp.full((16,), lvl, jnp.int32).astype(jnp.uint32)
                   << jnp.uint32(_LOG2_T))
        for v in range(_C // 16):
            xv = xyzb[pl.ds(v * 16, 16)]
            yv = xyzb[pl.ds(_C + v * 16, 16)]
            zv = xyzb[pl.ds(2 * _C + v * 16, 16)]
            px = ((xv + 1.0) * 0.5) * sv + 0.5
            py = ((yv + 1.0) * 0.5) * sv + 0.5
            pz = ((zv + 1.0) * 0.5) * sv + 0.5
            gx = px.astype(jnp.uint32)
            gy = py.astype(jnp.uint32)
            gz = pz.astype(jnp.uint32)
            wx1 = px - gx.astype(jnp.float32)
            wy1 = py - gy.astype(jnp.float32)
            wz1 = pz - gz.astype(jnp.float32)
            wx0 = 1.0 - wx1
            wy0 = 1.0 - wy1
            wz0 = 1.0 - wz1
            hy0 = gy * p2
            hy1 = hy0 + p2
            hz0 = gz * p3
            hz1 = hz0 + p3
            hxy = (gx ^ hy0, (gx + jnp.uint32(1)) ^ hy0,
                   gx ^ hy1, (gx + jnp.uint32(1)) ^ hy1)
            wxy = (wx0 * wy0, wx1 * wy0, wx0 * wy1, wx1 * wy1)
            mhbm = jnp.full((16,), m_not_sp, jnp.int32)
            for corner in range(8):
                hz = hz1 if (corner >> 2) & 1 else hz0
                wz = wz1 if (corner >> 2) & 1 else wz0
                eidx = (((hxy[corner & 3] ^ hz) & mask) | base_or)
                ei = eidx.astype(jnp.int32)
                mge = (jnp.int32(_HALF3 - 1) - ei) >> 31
                word = ei - (mge & jnp.int32(_HALF3))
                amt = jnp.int32(16) & ~mge
                idx = word ^ ((word ^ ei) & mhbm)
                o = p * _CB + corner * _C + v * 16
                idxb[pl.ds(o, 16)] = idx
                shb[pl.ds(o, 16)] = amt
                wb[pl.ds(o, 16)] = wxy[corner & 3] * wz
        issue(p, m_not_sp)

        @pl.when(k > 6)
        def _drain_prev():
            drain_accum(k - 7)

            @pl.when(jnp.logical_and(pos == 7, chunk > 0))
            def _flush_prev_chunk():
                pcp = (chunk - 1) & 1
                pltpu.sync_copy(
                    obuf.at[pl.ds(pcp * _C, _C)],
                    out_hbm.at[pl.ds(base + (chunk - 1) * _C, _C)])

        return carry

    total = _L * _CHUNKS
    lax.fori_loop(0, total, step, 0)
    for t in range(total - 7, total):
        drain_accum(t)
    pltpu.sync_copy(
        obuf.at[pl.ds(((_CHUNKS - 1) & 1) * _C, _C)],
        out_hbm.at[pl.ds(base + (_CHUNKS - 1) * _C, _C)])


_sc_kernel = functools.partial(
    pl.kernel,
    out_type=jax.ShapeDtypeStruct((_N_PTS,), jnp.float32),
    mesh=plsc.VectorSubcoreMesh(core_axis_name="c", subcore_axis_name="s"),
    scratch_types=[
        pltpu.VMEM((_C * 3,), jnp.float32),
        pltpu.VMEM((_L * 16,), jnp.float32),
        pltpu.VMEM((8 * _CB,), jnp.int32),
        pltpu.VMEM((8 * _CB,), jnp.int32),
        pltpu.VMEM((8 * _CB,), jnp.float32),
        pltpu.VMEM((8 * _CB,), jnp.float32),
        pltpu.VMEM((2 * _C,), jnp.float32),
        pltpu.VMEM_SHARED((_HALF3,), jnp.float32),
        pltpu.SemaphoreType.DMA((8,)),
    ],
)(_sc_body)


def _pack3_body(a_ref, b_ref, o_ref):
    lo = lax.bitcast_convert_type(
        a_ref[...].astype(jnp.bfloat16), jnp.uint16).astype(jnp.uint32)
    hi = lax.bitcast_convert_type(
        b_ref[...].astype(jnp.bfloat16), jnp.uint16).astype(jnp.uint32)
    o_ref[...] = lax.bitcast_convert_type(lo | (hi << 16), jnp.float32)


def _pack3(tsum):
    blk = 32768
    nb = _HALF3 // blk
    return pl.pallas_call(
        _pack3_body,
        grid=(nb,),
        in_specs=[pl.BlockSpec((blk,), lambda i: (i,)),
                  pl.BlockSpec((blk,), lambda i: (i + nb,))],
        out_specs=pl.BlockSpec((blk,), lambda i: (i,)),
        out_shape=jax.ShapeDtypeStruct((_HALF3,), jnp.float32),
    )(tsum, tsum)


def _level_scales():
    s = [_N_MIN * (_GROWTH ** lvl) - 1.0 for lvl in range(_L)]
    return np.repeat(np.asarray(s, np.float32)[:, None], 16, axis=1).reshape(-1)


def kernel(x, table):
    tview = table.reshape(_L, _T // 128, 128, _F).transpose(0, 1, 3, 2)
    tsum = _presum_table(tview)
    s_splat = jnp.asarray(_level_scales())
    xt = x.T
    sp3 = _pack3(tsum)
    return _sc_kernel(xt[0], xt[1], xt[2], tsum, sp3, s_splat)
